# Initial kernel scaffold; baseline (speedup 1.0000x reference)
import jax
import jax.numpy as jnp
from jax.experimental import pallas as pl


def kernel(x, S_indices, S_values, W, b):
    raise NotImplementedError("write your pallas kernel here")


# speedup vs baseline: 3.5942x; 3.5942x over previous
import functools

import jax
import jax.numpy as jnp
from jax import lax
from jax.experimental import pallas as pl
from jax.experimental.pallas import tpu as pltpu
from jax.experimental.pallas import tpu_sc as plsc

_L = 16
_NSUB = 16
_K = 128


def _mm_body(x_ref, w_ref, b_ref, o_ref):
    o_ref[...] = (
        jnp.dot(x_ref[...], w_ref[0], preferred_element_type=jnp.float32)
        + b_ref[0]
    )


def _matmul_split(x, W, b):
    N, F = x.shape
    C = W.shape[1]
    H = C // 2
    RB = 2000
    nr = N // RB
    W2 = jnp.moveaxis(W.reshape(F, 2, H), 1, 0)
    b2 = b.reshape(2, 1, H)
    return pl.pallas_call(
        _mm_body,
        grid=(nr, 2),
        in_specs=[
            pl.BlockSpec((RB, F), lambda i, j: (i, 0)),
            pl.BlockSpec((1, F, H), lambda i, j: (j, 0, 0)),
            pl.BlockSpec((1, 1, H), lambda i, j: (j, 0, 0)),
        ],
        out_specs=pl.BlockSpec((RB, H), lambda i, j: (j * nr + i, 0)),
        out_shape=jax.ShapeDtypeStruct((2 * N, H), jnp.float32),
    )(x, W2, b2)


def _make_sc_spmm(N, H, epw, nchunks):
    RPT = -(-(-(-N // _NSUB)) // 128) * 128
    N_pad = RPT * _NSUB
    ZB = 128
    assert RPT % ZB == 0
    mesh = plsc.VectorSubcoreMesh(core_axis_name="c", subcore_axis_name="s")

    @functools.partial(
        pl.kernel,
        mesh=mesh,
        compiler_params=pltpu.CompilerParams(use_tc_tiling_on_sc=False),
        out_type=jax.ShapeDtypeStruct((2 * N_pad, H), jnp.float32),
        scratch_types=[
            pltpu.VMEM((_K,), jnp.int32),
            pltpu.VMEM((_K,), jnp.int32),
            pltpu.VMEM((_K,), jnp.float32),
            pltpu.VMEM((_K, H), jnp.float32),
            pltpu.VMEM((ZB, H), jnp.float32),
            pltpu.VMEM_SHARED((N_pad, H), jnp.float32),
            pltpu.SemaphoreType.DMA,
        ],
    )
    def k(rows_hbm, cols_hbm, vals_hbm, h_hbm, out_hbm,
          rows_v, cols_v, vals_v, g_v, z_v, acc, sem):
        c = lax.axis_index("c")
        s = lax.axis_index("s")

        def _zrow(i, carry):
            for l in range(H // _L):
                z_v[i, pl.ds(l * _L, _L)] = jnp.zeros((_L,), jnp.float32)
            return carry

        lax.fori_loop(0, ZB, _zrow, 0)
        r0 = s * RPT
        for i in range(RPT // ZB):
            pltpu.sync_copy(z_v, acc.at[pl.ds(r0 + i * ZB, ZB)])
        plsc.subcore_barrier()

        ebase = s * epw
        col_off = jnp.full((_L,), c * N, jnp.int32)

        def _chunk(ki, carry):
            base = ebase + ki * _K
            pltpu.sync_copy(rows_hbm.at[pl.ds(base, _K)], rows_v)
            pltpu.sync_copy(cols_hbm.at[pl.ds(base, _K)], cols_v)
            pltpu.sync_copy(vals_hbm.at[pl.ds(base, _K)], vals_v)
            for j in range(_K // _L):
                cols_v[pl.ds(j * _L, _L)] = cols_v[pl.ds(j * _L, _L)] + col_off
            pltpu.async_copy(h_hbm.at[cols_v], g_v, sem).wait()
            for j16 in range(_K // _L):
                v16 = vals_v[pl.ds(j16 * _L, _L)]
                for j in range(_L):
                    sv = lax.gather(
                        v16,
                        jnp.full((_L, 1), j, jnp.int32),
                        lax.GatherDimensionNumbers(
                            offset_dims=(),
                            collapsed_slice_dims=(0,),
                            start_index_map=(0,),
                        ),
                        slice_sizes=(1,),
                        mode=lax.GatherScatterMode.PROMISE_IN_BOUNDS,
                    )
                    e = j16 * _L + j
                    for l in range(H // _L):
                        g_v[e, pl.ds(l * _L, _L)] = (
                            g_v[e, pl.ds(l * _L, _L)] * sv
                        )
            pltpu.sync_copy(g_v, acc.at[rows_v], add=True)
            return carry

        lax.fori_loop(0, nchunks, _chunk, 0)
        plsc.subcore_barrier()

        for i in range(RPT // ZB):
            pltpu.sync_copy(acc.at[pl.ds(r0 + i * ZB, ZB)], z_v)
            pltpu.sync_copy(
                z_v, out_hbm.at[pl.ds(c * N_pad + r0 + i * ZB, ZB)]
            )

    return k


def kernel(x, S_indices, S_values, W, b):
    N, _ = x.shape
    C = W.shape[1]
    H = C // 2
    E = S_values.shape[0]

    nchunks = -(-(-(-E // _NSUB)) // _K)
    epw = nchunks * _K
    pad = epw * _NSUB - E
    rows_p = jnp.pad(S_indices[0], (0, pad))
    cols_p = jnp.pad(S_indices[1], (0, pad))
    vals_p = jnp.pad(S_values, (0, pad))

    h2 = _matmul_split(x, W, b)
    out2 = _make_sc_spmm(N, H, epw, nchunks)(rows_p, cols_p, vals_p, h2)
    n_pad = out2.shape[0] // 2
    return jnp.concatenate([out2[:N], out2[n_pad:n_pad + N]], axis=1)


# speedup vs baseline: 5.1851x; 1.4426x over previous
import functools

import jax
import jax.numpy as jnp
from jax import lax
from jax.experimental import pallas as pl
from jax.experimental.pallas import tpu as pltpu
from jax.experimental.pallas import tpu_sc as plsc

_L = 16
_NSUB = 16
_K = 128


def _mm_body(x_ref, w_ref, b_ref, o_ref):
    o_ref[...] = (
        jnp.dot(x_ref[...], w_ref[0], preferred_element_type=jnp.float32)
        + b_ref[0]
    )


def _matmul_split(x, W, b):
    N, F = x.shape
    C = W.shape[1]
    H = C // 2
    RB = 2000
    nr = N // RB
    W2 = jnp.moveaxis(W.reshape(F, 2, H), 1, 0)
    b2 = b.reshape(2, 1, H)
    return pl.pallas_call(
        _mm_body,
        grid=(nr, 2),
        in_specs=[
            pl.BlockSpec((RB, F), lambda i, j: (i, 0)),
            pl.BlockSpec((1, F, H), lambda i, j: (j, 0, 0)),
            pl.BlockSpec((1, 1, H), lambda i, j: (j, 0, 0)),
        ],
        out_specs=pl.BlockSpec((RB, H), lambda i, j: (j * nr + i, 0)),
        out_shape=jax.ShapeDtypeStruct((2 * N, H), jnp.float32),
    )(x, W2, b2)


def _splat_lane(v16, j):
    return lax.gather(
        v16,
        jnp.full((_L, 1), j, jnp.int32),
        lax.GatherDimensionNumbers(
            offset_dims=(),
            collapsed_slice_dims=(0,),
            start_index_map=(0,),
        ),
        slice_sizes=(1,),
        mode=lax.GatherScatterMode.PROMISE_IN_BOUNDS,
    )


def _make_sc_spmm(N, H, nchunks):
    RPT = -(-(-(-N // _NSUB)) // 128) * 128
    N_pad = RPT * _NSUB
    ZB = 128
    assert RPT % ZB == 0
    assert nchunks >= 4 and nchunks % 2 == 0
    npairs = (nchunks - 2) // 2
    mesh = plsc.VectorSubcoreMesh(core_axis_name="c", subcore_axis_name="s")

    @functools.partial(
        pl.kernel,
        mesh=mesh,
        compiler_params=pltpu.CompilerParams(use_tc_tiling_on_sc=False),
        out_type=jax.ShapeDtypeStruct((2 * N_pad, H), jnp.float32),
        scratch_types=[
            pltpu.VMEM((2, _K), jnp.int32),
            pltpu.VMEM((2, _K), jnp.int32),
            pltpu.VMEM((_K,), jnp.float32),
            pltpu.VMEM((_K,), jnp.float32),
            pltpu.VMEM((_K, H), jnp.float32),
            pltpu.VMEM((_K, H), jnp.float32),
            pltpu.VMEM((ZB, H), jnp.float32),
            pltpu.VMEM_SHARED((N_pad, H), jnp.float32),
            pltpu.SemaphoreType.DMA,
            pltpu.SemaphoreType.DMA,
            pltpu.SemaphoreType.DMA,
            pltpu.SemaphoreType.DMA,
        ],
    )
    def k(p_hbm, v_hbm, h_hbm, out_hbm,
          ib0, ib1, vb0, vb1, g0, g1, z_v, acc, gs0, gs1, ss0, ss1):
        c = lax.axis_index("c")
        s = lax.axis_index("s")
        ib = (ib0, ib1)
        vb = (vb0, vb1)
        g = (g0, g1)
        gs = (gs0, gs1)
        ss = (ss0, ss1)

        def _zrow(i, carry):
            for l in range(H // _L):
                z_v[i, pl.ds(l * _L, _L)] = jnp.zeros((_L,), jnp.float32)
            return carry

        lax.fori_loop(0, ZB, _zrow, 0)
        r0 = s * RPT
        for i in range(RPT // ZB):
            pltpu.sync_copy(z_v, acc.at[pl.ds(r0 + i * ZB, ZB)])
        plsc.subcore_barrier()

        pbase = (c * _NSUB + s) * nchunks
        vbase = s * nchunks

        def load_and_gather(ki, b):
            pltpu.sync_copy(p_hbm.at[pbase + ki], ib[b])
            pltpu.sync_copy(v_hbm.at[vbase + ki], vb[b])
            pltpu.async_copy(h_hbm.at[ib[b].at[1]], g[b], gs[b])

        def wait_scatter(b):
            pltpu.make_async_copy(g[b], acc.at[ib[b].at[0]], ss[b]).wait()

        def scale_and_scatter(b):
            pltpu.make_async_copy(h_hbm.at[ib[b].at[1]], g[b], gs[b]).wait()
            for j16 in range(_K // _L):
                v16 = vb[b][pl.ds(j16 * _L, _L)]
                for j in range(_L):
                    sv = _splat_lane(v16, j)
                    e = j16 * _L + j
                    for l in range(H // _L):
                        g[b][e, pl.ds(l * _L, _L)] = (
                            g[b][e, pl.ds(l * _L, _L)] * sv
                        )
            pltpu.async_copy(g[b], acc.at[ib[b].at[0]], ss[b], add=True)

        load_and_gather(0, 0)
        load_and_gather(1, 1)
        scale_and_scatter(0)

        def _pair(p, carry):
            ki = 2 * p + 1
            wait_scatter(0)
            load_and_gather(ki + 1, 0)
            scale_and_scatter(1)
            wait_scatter(1)
            load_and_gather(ki + 2, 1)
            scale_and_scatter(0)
            return carry

        lax.fori_loop(0, npairs, _pair, 0)
        scale_and_scatter(1)
        wait_scatter(0)
        wait_scatter(1)
        plsc.subcore_barrier()

        for i in range(RPT // ZB):
            pltpu.sync_copy(acc.at[pl.ds(r0 + i * ZB, ZB)], z_v)
            pltpu.sync_copy(
                z_v, out_hbm.at[pl.ds(c * N_pad + r0 + i * ZB, ZB)]
            )

    return k


def kernel(x, S_indices, S_values, W, b):
    N, _ = x.shape
    C = W.shape[1]
    H = C // 2
    E = S_values.shape[0]

    nchunks = -(-(-(-E // _NSUB)) // _K)
    nchunks = max(4, nchunks + (nchunks % 2))
    epw = nchunks * _K
    pad = epw * _NSUB - E
    rows_p = jnp.pad(S_indices[0], (0, pad)).reshape(_NSUB * nchunks, _K)
    cols_p = jnp.pad(S_indices[1], (0, pad)).reshape(_NSUB * nchunks, _K)
    vals_p = jnp.pad(S_values, (0, pad)).reshape(_NSUB * nchunks, _K)
    packed = jnp.stack(
        [
            jnp.stack([rows_p, cols_p + c * N], axis=1)
            for c in range(2)
        ],
        axis=0,
    ).reshape(2 * _NSUB * nchunks, 2, _K)

    h2 = _matmul_split(x, W, b)
    out2 = _make_sc_spmm(N, H, nchunks)(packed, vals_p, h2)
    n_pad = out2.shape[0] // 2
    return jnp.concatenate([out2[:N], out2[n_pad:n_pad + N]], axis=1)
